# baseline (speedup 1.0000x reference)
import functools

import jax
import jax.numpy as jnp
from jax import lax
from jax.experimental import pallas as pl
from jax.experimental.pallas import tpu as pltpu
from jax.experimental.pallas import tpu_sc as plsc

N = 10000
E = 160000
H = 256
L = 6
G = 512
HH = H // 2
NSUB = 16
K = 80
RP = 10112
RPT = RP // NSUB
NEP = 10240


def _make_sc_scatter(n_items, do_edge, apply_relu):
  per_sub = n_items // NSUB
  nchunks = per_sub // K
  assert per_sub % K == 0

  mesh = plsc.VectorSubcoreMesh(core_axis_name="c", subcore_axis_name="s")
  scratch = [
      pltpu.VMEM((K,), jnp.int32),
      pltpu.VMEM((K,), jnp.int32),
      pltpu.VMEM((K,), jnp.int32),
      pltpu.VMEM((K, HH), jnp.float32),
      pltpu.VMEM((K, HH), jnp.float32),
      pltpu.VMEM_SHARED((RP, HH), jnp.float32),
      pltpu.SemaphoreType.DMA,
  ]

  @functools.partial(
      pl.kernel,
      mesh=mesh,
      out_type=jax.ShapeDtypeStruct((2 * RP, HH), jnp.float32),
      scratch_types=scratch,
  )
  def kfn(xlo, xhi, elo, ehi, idx_h, attr_h, dst_h, zeros_h, out,
          iv, av, dv, xr, er, acc, sem):
    cid = lax.axis_index("c")
    sid = lax.axis_index("s")

    pltpu.sync_copy(zeros_h.at[pl.ds(sid * RPT, RPT)],
                    acc.at[pl.ds(sid * RPT, RPT)])
    plsc.subcore_barrier()

    base = sid * per_sub

    def chunk(i, carry):
      off = base + i * K
      pltpu.sync_copy(idx_h.at[pl.ds(off, K)], iv)
      if do_edge:
        pltpu.sync_copy(attr_h.at[pl.ds(off, K)], av)
      pltpu.sync_copy(dst_h.at[pl.ds(off, K)], dv)

      @pl.when(cid == 0)
      def _():
        pltpu.async_copy(xlo.at[iv], xr, sem).wait()
        if do_edge:
          pltpu.async_copy(elo.at[av], er, sem).wait()

      @pl.when(cid == 1)
      def _():
        pltpu.async_copy(xhi.at[iv], xr, sem).wait()
        if do_edge:
          pltpu.async_copy(ehi.at[av], er, sem).wait()

      if do_edge:
        def row(r, c2):
          for j in range(HH // 16):
            v = xr[r, pl.ds(j * 16, 16)] + er[r, pl.ds(j * 16, 16)]
            if apply_relu:
              v = jnp.maximum(v, 0.0)
            xr[r, pl.ds(j * 16, 16)] = v
          return c2
        lax.fori_loop(0, K, row, 0)

      pltpu.sync_copy(xr, acc.at[dv], add=True)
      return carry

    lax.fori_loop(0, nchunks, chunk, 0)
    plsc.subcore_barrier()

    pltpu.sync_copy(acc.at[pl.ds(sid * RPT, RPT)],
                    out.at[pl.ds(cid * RP + sid * RPT, RPT)])

  return kfn


def _mlp_block(z_ref, xi_ref, w1_ref, w2_ref, c1_ref, d1_ref, c2_ref, d2_ref,
               out_ref):
  z = z_ref[...]
  h = jnp.dot(z, w1_ref[...], preferred_element_type=jnp.float32)
  h = jnp.maximum(h * c1_ref[...] + d1_ref[...], 0.0)
  h2 = jnp.dot(h, w2_ref[...], preferred_element_type=jnp.float32)
  r = xi_ref[...] + h2
  out_ref[...] = jnp.maximum(r * c2_ref[...] + d2_ref[...], 0.0)


_MLP_BLK = 400
_MLP_GRID = N // _MLP_BLK


def _tc_mlp(z, xi, w1, w2, c1, d1, c2, d2):
  row_spec = pl.BlockSpec((_MLP_BLK, H), lambda i: (i, 0))
  full = pl.BlockSpec((H, H), lambda i: (0, 0))
  vec = pl.BlockSpec((1, H), lambda i: (0, 0))
  return pl.pallas_call(
      _mlp_block,
      grid=(_MLP_GRID,),
      in_specs=[row_spec, row_spec, full, full, vec, vec, vec, vec],
      out_specs=row_spec,
      out_shape=jax.ShapeDtypeStruct((N, H), jnp.float32),
  )(z, xi, w1, w2, c1, d1, c2, d2)


def _pool_body(batch_ref, x_ref, rw1m_ref, rw1s_ref, rw1x_ref, rb1_ref,
               rw2_ref, rb2_ref, out_ref, sums, mx, cnt):
  sums[...] = jnp.zeros_like(sums)
  mx[...] = jnp.full_like(mx, -jnp.inf)
  cnt[...] = jnp.zeros_like(cnt)

  def body(n, carry):
    g = batch_ref[n]
    row = x_ref[pl.ds(n, 1), :]
    sums[pl.ds(g, 1), :] += row
    mx[pl.ds(g, 1), :] = jnp.maximum(mx[pl.ds(g, 1), :], row)
    cnt[pl.ds(g, 1), :] += 1.0
    return carry

  lax.fori_loop(0, N, body, 0)

  s = sums[...]
  c = jnp.maximum(cnt[:, :1], 1.0)
  mean = s / c
  m = mx[...]
  hreg = (jnp.dot(mean, rw1m_ref[...], preferred_element_type=jnp.float32)
          + jnp.dot(s, rw1s_ref[...], preferred_element_type=jnp.float32)
          + jnp.dot(m, rw1x_ref[...], preferred_element_type=jnp.float32)
          + rb1_ref[...])
  hreg = jnp.maximum(hreg, 0.0)
  o = jnp.sum(hreg * rw2_ref[...], axis=1, keepdims=True) + rb2_ref[0, 0]
  out_ref[...] = jnp.broadcast_to(o, out_ref.shape)


def _tc_pool_readout(xi, batch, rw1, rb1, rw2, rb2):
  rw1m = rw1[:H]
  rw1s = rw1[H:2 * H]
  rw1x = rw1[2 * H:]
  rb1b = rb1.reshape(1, H)
  rw2r = rw2[:, 0].reshape(1, H)
  rb2b = jnp.broadcast_to(rb2.reshape(1, 1), (1, 1))
  hmat = pl.BlockSpec((H, H), lambda: (0, 0))
  res = pl.pallas_call(
      _pool_body,
      in_specs=[
          pl.BlockSpec(memory_space=pltpu.SMEM),
          pl.BlockSpec((N, H), lambda: (0, 0)),
          hmat, hmat, hmat,
          pl.BlockSpec((1, H), lambda: (0, 0)),
          pl.BlockSpec((1, H), lambda: (0, 0)),
          pl.BlockSpec((1, 1), memory_space=pltpu.SMEM),
      ],
      out_specs=pl.BlockSpec((G, 128), lambda: (0, 0)),
      out_shape=jax.ShapeDtypeStruct((G, 128), jnp.float32),
      scratch_shapes=[
          pltpu.VMEM((G, H), jnp.float32),
          pltpu.VMEM((G, H), jnp.float32),
          pltpu.VMEM((G, 128), jnp.float32),
      ],
  )(batch, xi, rw1m, rw1s, rw1x, rb1b, rw2r, rb2b)
  return res[:, 0]


_sc_embed = _make_sc_scatter(NEP, do_edge=False, apply_relu=False)
_sc_msg = _make_sc_scatter(E, do_edge=True, apply_relu=True)


def kernel(x, edge_attr, edge_index, batch, node_table, edge_table,
           W1, b1, bn1_g, bn1_b, W2, b2, eps, bn_g, bn_b,
           rW1, rb1, rW2, rb2):
  inv = 1.0 / jnp.sqrt(1.0 + 1e-5)
  src = edge_index[0].astype(jnp.int32)
  dst = edge_index[1].astype(jnp.int32)
  attr = edge_attr[:, 0].astype(jnp.int32)
  batch = batch.astype(jnp.int32)
  zeros = jnp.zeros((RP, HH), jnp.float32)
  ez = jnp.zeros((4, HH), jnp.float32)

  xidx = jnp.concatenate(
      [x[:, 0].astype(jnp.int32), jnp.zeros((NEP - N,), jnp.int32)])
  ndst = jnp.concatenate(
      [jnp.arange(N, dtype=jnp.int32),
       jnp.full((NEP - N,), N, jnp.int32)])
  nzi = jnp.zeros((NEP,), jnp.int32)
  emb = _sc_embed(node_table[:, :HH], node_table[:, HH:], ez, ez,
                  xidx, nzi, ndst, zeros)
  xi = jnp.concatenate([emb[:N], emb[RP:RP + N]], axis=1)

  etlo = edge_table[:, :HH]
  ethi = edge_table[:, HH:]
  for l in range(L):
    aggf = _sc_msg(xi[:, :HH], xi[:, HH:], etlo, ethi, src, attr, dst, zeros)
    agg = jnp.concatenate([aggf[:N], aggf[RP:RP + N]], axis=1)
    z = (1.0 + eps[l]) * xi + agg
    c1 = (bn1_g[l] * inv).reshape(1, H)
    d1 = (bn1_g[l] * inv * b1[l] + bn1_b[l]).reshape(1, H)
    c2 = (bn_g[l] * inv).reshape(1, H)
    d2 = (bn_g[l] * inv * b2[l] + bn_b[l]).reshape(1, H)
    xi = _tc_mlp(z, xi, W1[l], W2[l], c1, d1, c2, d2)

  out = _tc_pool_readout(xi, batch, rW1, rb1, rW2, rb2)
  return (out, xi)
